# Initial kernel scaffold; baseline (speedup 1.0000x reference)
import jax
import jax.numpy as jnp
from jax.experimental import pallas as pl


def kernel(x, ei, pos, neg, W1, b1, W2, b2, W3, b3):
    raise NotImplementedError("write your pallas kernel here")


# speedup vs baseline: 2.4456x; 2.4456x over previous
import functools

import jax
import jax.numpy as jnp
from jax.experimental import pallas as pl
from jax.experimental.pallas import tpu as pltpu


def _loss_body(za_ref, zb_ref, nfirst_ref, out_ref, acc_ref):
    i = pl.program_id(0)
    za = za_ref[...]
    zb = zb_ref[...]
    preds = jnp.sum(za * zb, axis=1, keepdims=True) + nfirst_ref[1]
    sp = jnp.maximum(preds, 0.0) + jnp.log1p(jnp.exp(-jnp.abs(preds)))
    base = i * za.shape[0]
    rows = base + jax.lax.broadcasted_iota(jnp.int32, preds.shape, 0)
    labeled = (rows < nfirst_ref[0]).astype(jnp.float32)
    contrib = jnp.sum(sp - labeled * preds)

    @pl.when(i == 0)
    def _init():
        acc_ref[0] = 0.0

    acc_ref[0] = acc_ref[0] + contrib

    @pl.when(i == pl.num_programs(0) - 1)
    def _fin():
        out_ref[0] = acc_ref[0]


def _loss_reduce(za, zb, nfirst, b3, block=1000):
    m = za.shape[0]
    grid = m // block
    nf = jnp.stack([jnp.float32(nfirst), b3[0]])
    out = pl.pallas_call(
        _loss_body,
        grid=(grid,),
        in_specs=[
            pl.BlockSpec((block, za.shape[1]), lambda i: (i, 0)),
            pl.BlockSpec((block, za.shape[1]), lambda i: (i, 0)),
            pl.BlockSpec(memory_space=pltpu.SMEM),
        ],
        out_specs=pl.BlockSpec(memory_space=pltpu.SMEM),
        out_shape=jax.ShapeDtypeStruct((1,), jnp.float32),
        scratch_shapes=[pltpu.SMEM((1,), jnp.float32)],
    )(za, zb, nf)
    return out[0] / m


def kernel(x, ei, pos, neg, W1, b1, W2, b2, W3, b3):
    n = x.shape[0]
    src, dst = ei[0], ei[1]
    deg = jnp.ones((n,), jnp.float32).at[dst].add(1.0)
    dis = jax.lax.rsqrt(deg)

    xp = x * dis[:, None]
    u = xp @ W1
    s = u.at[dst].add(u[src])
    h = jax.nn.relu(dis[:, None] * s + b1)

    g = (h * dis[:, None]) @ W2
    t = g.at[dst].add(g[src])
    z = dis[:, None] * t + b2

    zw = z * W3[:, 0][None, :]
    za = jnp.concatenate([z[pos[0]], z[neg[0]]], axis=0)
    zb = jnp.concatenate([zw[pos[1]], zw[neg[1]]], axis=0)
    return _loss_reduce(za, zb, pos.shape[0], b3)


# speedup vs baseline: 8.5129x; 3.4809x over previous
import functools

import jax
import jax.numpy as jnp
from jax import lax
from jax.experimental import pallas as pl
from jax.experimental.pallas import tpu as pltpu
from jax.experimental.pallas import tpu_sc as plsc

N = 10000
NP = 10240
E = 320000
P2 = 200000
PP = 204800
NC = 2
NS = 16
NW = NC * NS
EPW = E // NW
ECH = 80
PPW = PP // NW
PCH = 128

_mesh = plsc.VectorSubcoreMesh(core_axis_name="c", subcore_axis_name="s")
_f32 = jnp.float32


def _wid():
    return lax.axis_index("s") * NC + lax.axis_index("c")


def _deg_body(dst_hbm, z1_hbm, out_hbm, idx_v, ones_v, acc_sh, sem):
    c = lax.axis_index("c")
    s = lax.axis_index("s")
    w = _wid()
    stripe = NP // NS
    pltpu.sync_copy(z1_hbm.at[pl.ds(s * stripe, stripe)],
                    acc_sh.at[pl.ds(s * stripe, stripe)])
    for i in range(ECH // 16):
        ones_v[pl.ds(i * 16, 16)] = jnp.ones((16,), _f32)
    plsc.subcore_barrier()

    def body(j, carry):
        base = w * EPW + j * ECH
        pltpu.sync_copy(dst_hbm.at[pl.ds(base, ECH)], idx_v)
        pltpu.sync_copy(ones_v, acc_sh.at[idx_v], add=True)
        return carry

    lax.fori_loop(0, EPW // ECH, body, 0)
    plsc.subcore_barrier()
    pltpu.sync_copy(acc_sh.at[pl.ds(s * stripe, stripe)],
                    out_hbm.at[c, pl.ds(s * stripe, stripe)])


_deg_kernel = functools.partial(
    pl.kernel, _deg_body,
    out_type=jax.ShapeDtypeStruct((NC, NP), _f32),
    mesh=_mesh,
    scratch_types=[
        pltpu.VMEM((ECH,), jnp.int32),
        pltpu.VMEM((ECH,), _f32),
        pltpu.VMEM_SHARED((NP,), _f32),
        pltpu.SemaphoreType.DMA,
    ],
)()


def _agg_body(u_hbm, src_hbm, dst_hbm, zw_hbm, out_hbm,
              si_v, di_v, rows_v, acc_sh, sem):
    c = lax.axis_index("c")
    s = lax.axis_index("s")
    w = _wid()
    stripe = NP // NS
    pltpu.sync_copy(zw_hbm.at[pl.ds(s * stripe, stripe)],
                    acc_sh.at[pl.ds(s * stripe, stripe)])
    plsc.subcore_barrier()

    def body(j, carry):
        base = w * EPW + j * ECH
        pltpu.sync_copy(src_hbm.at[pl.ds(base, ECH)], si_v)
        pltpu.sync_copy(dst_hbm.at[pl.ds(base, ECH)], di_v)
        pltpu.async_copy(u_hbm.at[si_v], rows_v, sem).wait()
        pltpu.sync_copy(rows_v, acc_sh.at[di_v], add=True)
        return carry

    lax.fori_loop(0, EPW // ECH, body, 0)
    plsc.subcore_barrier()
    pltpu.sync_copy(acc_sh.at[pl.ds(s * stripe, stripe)],
                    out_hbm.at[c, pl.ds(s * stripe, stripe)])


def _make_agg(width):
    return functools.partial(
        pl.kernel, _agg_body,
        out_type=jax.ShapeDtypeStruct((NC, NP, width), _f32),
        mesh=_mesh,
        compiler_params=pltpu.CompilerParams(use_tc_tiling_on_sc=False),
        scratch_types=[
            pltpu.VMEM((ECH,), jnp.int32),
            pltpu.VMEM((ECH,), jnp.int32),
            pltpu.VMEM((ECH, width), _f32),
            pltpu.VMEM_SHARED((NP, width), _f32),
            pltpu.SemaphoreType.DMA,
        ],
    )()


_agg128 = _make_agg(128)
_agg64 = _make_agg(64)


def _pair_body(z_hbm, zw_hbm, ai_hbm, bi_hbm, za_hbm, zb_hbm,
               ia_v, ib_v, ra_v, rb_v, sem):
    w = _wid()

    def body(j, carry):
        base = w * PPW + j * PCH
        pltpu.sync_copy(ai_hbm.at[pl.ds(base, PCH)], ia_v)
        pltpu.sync_copy(bi_hbm.at[pl.ds(base, PCH)], ib_v)
        pltpu.async_copy(z_hbm.at[ia_v], ra_v, sem).wait()
        pltpu.sync_copy(ra_v, za_hbm.at[pl.ds(base, PCH)])
        pltpu.async_copy(zw_hbm.at[ib_v], rb_v, sem).wait()
        pltpu.sync_copy(rb_v, zb_hbm.at[pl.ds(base, PCH)])
        return carry

    lax.fori_loop(0, PPW // PCH, body, 0)


_pair_kernel = functools.partial(
    pl.kernel, _pair_body,
    out_type=(jax.ShapeDtypeStruct((PP, 64), _f32),
              jax.ShapeDtypeStruct((PP, 64), _f32)),
    mesh=_mesh,
    compiler_params=pltpu.CompilerParams(use_tc_tiling_on_sc=False),
    scratch_types=[
        pltpu.VMEM((PCH,), jnp.int32),
        pltpu.VMEM((PCH,), jnp.int32),
        pltpu.VMEM((PCH, 64), _f32),
        pltpu.VMEM((PCH, 64), _f32),
        pltpu.SemaphoreType.DMA,
    ],
)()


def _dis(degp_ref):
    deg = degp_ref[0, :] + degp_ref[1, :] + 1.0
    return lax.rsqrt(deg)


def _prescale_body(x_ref, degp_ref, xp_ref):
    xp_ref[...] = x_ref[...] * _dis(degp_ref)[:, None]


def _prescale(x, degp, blk=1024):
    return pl.pallas_call(
        _prescale_body,
        grid=(pl.cdiv(N, blk),),
        in_specs=[
            pl.BlockSpec((blk, 128), lambda i: (i, 0)),
            pl.BlockSpec((NC, blk), lambda i: (0, i)),
        ],
        out_specs=pl.BlockSpec((blk, 128), lambda i: (i, 0)),
        out_shape=jax.ShapeDtypeStruct((N, 128), _f32),
    )(x, degp)


def _layer1_body(sp_ref, xp_ref, degp_ref, b1_ref, w1_ref, w2_ref, g_ref):
    dis = _dis(degp_ref)[:, None]
    m = (sp_ref[0] + sp_ref[1] + xp_ref[...]) * dis
    h = jax.nn.relu(jnp.dot(m, w1_ref[...],
                            preferred_element_type=_f32) + b1_ref[...])
    g_ref[...] = jnp.dot(h * dis, w2_ref[...], preferred_element_type=_f32)


def _layer1(sp, xp, degp, b1, W1, W2, blk=1024):
    return pl.pallas_call(
        _layer1_body,
        grid=(pl.cdiv(N, blk),),
        in_specs=[
            pl.BlockSpec((NC, blk, 128), lambda i: (0, i, 0)),
            pl.BlockSpec((blk, 128), lambda i: (i, 0)),
            pl.BlockSpec((NC, blk), lambda i: (0, i)),
            pl.BlockSpec((1, 128), lambda i: (0, 0)),
            pl.BlockSpec((128, 128), lambda i: (0, 0)),
            pl.BlockSpec((128, 64), lambda i: (0, 0)),
        ],
        out_specs=pl.BlockSpec((blk, 64), lambda i: (i, 0)),
        out_shape=jax.ShapeDtypeStruct((N, 64), _f32),
    )(sp, xp, degp, b1, W1, W2)


def _layer2_body(tp_ref, g_ref, degp_ref, b2_ref, w3_ref, z_ref, zw_ref):
    dis = _dis(degp_ref)[:, None]
    z = (tp_ref[0] + tp_ref[1] + g_ref[...]) * dis + b2_ref[...]
    z_ref[...] = z
    zw_ref[...] = z * w3_ref[...]


def _layer2(tp, g, degp, b2, w3row, blk=1024):
    return pl.pallas_call(
        _layer2_body,
        grid=(pl.cdiv(N, blk),),
        in_specs=[
            pl.BlockSpec((NC, blk, 64), lambda i: (0, i, 0)),
            pl.BlockSpec((blk, 64), lambda i: (i, 0)),
            pl.BlockSpec((NC, blk), lambda i: (0, i)),
            pl.BlockSpec((1, 64), lambda i: (0, 0)),
            pl.BlockSpec((1, 64), lambda i: (0, 0)),
        ],
        out_specs=[
            pl.BlockSpec((blk, 64), lambda i: (i, 0)),
            pl.BlockSpec((blk, 64), lambda i: (i, 0)),
        ],
        out_shape=[
            jax.ShapeDtypeStruct((N, 64), _f32),
            jax.ShapeDtypeStruct((N, 64), _f32),
        ],
    )(tp, g, degp, b2, w3row)


def _loss_body(za_ref, zb_ref, sc_ref, out_ref, acc_ref):
    i = pl.program_id(0)
    preds = jnp.sum(za_ref[...] * zb_ref[...], axis=1, keepdims=True) + sc_ref[1]
    sp = jnp.maximum(preds, 0.0) + jnp.log1p(jnp.exp(-jnp.abs(preds)))
    base = i * za_ref.shape[0]
    rows = base + lax.broadcasted_iota(jnp.int32, preds.shape, 0)
    labeled = (rows < 2).astype(_f32)
    contrib = jnp.sum(sp - labeled * preds)

    @pl.when(i == 0)
    def _init():
        acc_ref[0] = 0.0

    acc_ref[0] = acc_ref[0] + contrib

    @pl.when(i == pl.num_programs(0) - 1)
    def _fin():
        out_ref[0] = acc_ref[0]


def _loss_reduce(za, zb, b3, blk=1000):
    sc = jnp.stack([jnp.float32(0.0), b3[0]])
    out = pl.pallas_call(
        _loss_body,
        grid=(P2 // blk,),
        in_specs=[
            pl.BlockSpec((blk, 64), lambda i: (i, 0)),
            pl.BlockSpec((blk, 64), lambda i: (i, 0)),
            pl.BlockSpec(memory_space=pltpu.SMEM),
        ],
        out_specs=pl.BlockSpec(memory_space=pltpu.SMEM),
        out_shape=jax.ShapeDtypeStruct((1,), _f32),
        scratch_shapes=[pltpu.SMEM((1,), _f32)],
    )(za, zb, sc)
    return out[0] / P2


def kernel(x, ei, pos, neg, W1, b1, W2, b2, W3, b3):
    src, dst = ei[0], ei[1]
    padidx = jnp.arange(PP - P2, dtype=jnp.int32) % N
    ai = jnp.concatenate([pos[0], neg[0], padidx])
    bi = jnp.concatenate([pos[1], neg[1], padidx])
    z1 = jnp.zeros((NP,), _f32)
    z128 = jnp.zeros((NP, 128), _f32)
    z64 = jnp.zeros((NP, 64), _f32)

    degp = _deg_kernel(dst, z1)
    xp = _prescale(x, degp)
    sp = _agg128(xp, src, dst, z128)
    g = _layer1(sp, xp, degp, b1.reshape(1, 128), W1, W2)
    tp = _agg64(g, src, dst, z64)
    z, zw = _layer2(tp, g, degp, b2.reshape(1, 64), W3.reshape(1, 64))
    za, zb = _pair_kernel(z, zw, ai, bi)
    return _loss_reduce(za, zb, b3)


# speedup vs baseline: 12.7399x; 1.4965x over previous
import functools

import jax
import jax.numpy as jnp
from jax import lax
from jax.experimental import pallas as pl
from jax.experimental.pallas import tpu as pltpu
from jax.experimental.pallas import tpu_sc as plsc

N = 10000
NP = 10240
E = 320000
P2 = 200000
NC = 2
NS = 16
NW = NC * NS
EPW = E // NW
ECH = 80
ENCH = EPW // ECH
PCH = 128
PRC = 50
PNCH = PRC + 1
PP = NW * PCH * PRC

_mesh = plsc.VectorSubcoreMesh(core_axis_name="c", subcore_axis_name="s")
_params = pltpu.CompilerParams(use_tc_tiling_on_sc=False)
_f32 = jnp.float32


def _wid():
    return lax.axis_index("s") * NC + lax.axis_index("c")


def _deg_body(dst_hbm, z1_hbm, out_hbm, di_v, ones_v, acc_sh, sem):
    c = lax.axis_index("c")
    s = lax.axis_index("s")
    stripe = NP // NS
    pltpu.sync_copy(z1_hbm.at[pl.ds(s * stripe, stripe)],
                    acc_sh.at[pl.ds(s * stripe, stripe)])
    pltpu.sync_copy(dst_hbm.at[_wid()], di_v)
    for i in range(ECH // 16):
        ones_v[pl.ds(i * 16, 16)] = jnp.ones((16,), _f32)
    plsc.subcore_barrier()

    def body(i, carry):
        for k in range(25):
            pltpu.async_copy(ones_v, acc_sh.at[di_v.at[i * 25 + k]], sem,
                             add=True)
        for k in range(25):
            pltpu.make_async_copy(ones_v, acc_sh.at[di_v.at[i * 25 + k]],
                                  sem).wait()
        return carry

    lax.fori_loop(0, ENCH // 25, body, 0)
    plsc.subcore_barrier()
    pltpu.sync_copy(acc_sh.at[pl.ds(s * stripe, stripe)],
                    out_hbm.at[c, pl.ds(s * stripe, stripe)])


_deg_kernel = functools.partial(
    pl.kernel, _deg_body,
    out_type=jax.ShapeDtypeStruct((NC, NP), _f32),
    mesh=_mesh,
    compiler_params=_params,
    scratch_types=[
        pltpu.VMEM((ENCH, ECH), jnp.int32),
        pltpu.VMEM((ECH,), _f32),
        pltpu.VMEM_SHARED((NP,), _f32),
        pltpu.SemaphoreType.DMA,
    ],
)()


def _agg_body(u_hbm, src_hbm, dst_hbm, zw_hbm, out_hbm,
              si_v, di_v, r0, r1, acc_sh, sem0, sem1):
    c = lax.axis_index("c")
    s = lax.axis_index("s")
    w = _wid()
    stripe = NP // NS
    pltpu.sync_copy(zw_hbm.at[pl.ds(s * stripe, stripe)],
                    acc_sh.at[pl.ds(s * stripe, stripe)])
    pltpu.sync_copy(src_hbm.at[w], si_v)
    pltpu.sync_copy(dst_hbm.at[w], di_v)
    plsc.subcore_barrier()

    pltpu.async_copy(u_hbm.at[si_v.at[0]], r0, sem0)

    def body(i, carry):
        j0 = 2 * i
        pltpu.make_async_copy(u_hbm.at[si_v.at[j0]], r0, sem0).wait()
        pltpu.async_copy(u_hbm.at[si_v.at[j0 + 1]], r1, sem1)
        pltpu.sync_copy(r0, acc_sh.at[di_v.at[j0]], add=True)
        pltpu.make_async_copy(u_hbm.at[si_v.at[j0 + 1]], r1, sem1).wait()
        pltpu.async_copy(u_hbm.at[si_v.at[j0 + 2]], r0, sem0)
        pltpu.sync_copy(r1, acc_sh.at[di_v.at[j0 + 1]], add=True)
        return carry

    lax.fori_loop(0, (ENCH - 1) // 2, body, 0)
    pltpu.make_async_copy(u_hbm.at[si_v.at[ENCH - 1]], r0, sem0).wait()
    pltpu.sync_copy(r0, acc_sh.at[di_v.at[ENCH - 1]], add=True)
    plsc.subcore_barrier()
    pltpu.sync_copy(acc_sh.at[pl.ds(s * stripe, stripe)],
                    out_hbm.at[c, pl.ds(s * stripe, stripe)])


def _make_agg(width):
    return functools.partial(
        pl.kernel, _agg_body,
        out_type=jax.ShapeDtypeStruct((NC, NP, width), _f32),
        mesh=_mesh,
        compiler_params=_params,
        scratch_types=[
            pltpu.VMEM((ENCH, ECH), jnp.int32),
            pltpu.VMEM((ENCH, ECH), jnp.int32),
            pltpu.VMEM((ECH, width), _f32),
            pltpu.VMEM((ECH, width), _f32),
            pltpu.VMEM_SHARED((NP, width), _f32),
            pltpu.SemaphoreType.DMA,
            pltpu.SemaphoreType.DMA,
        ],
    )()


_agg128 = _make_agg(128)
_agg64 = _make_agg(64)


def _pair_body(z_hbm, zw_hbm, ai_hbm, bi_hbm, za_hbm, zb_hbm,
               ia_v, ib_v, ra, rb, semA, semB):
    w = _wid()
    pltpu.sync_copy(ai_hbm.at[w], ia_v)
    pltpu.sync_copy(bi_hbm.at[w], ib_v)

    pltpu.async_copy(z_hbm.at[ia_v.at[0]], ra, semA)

    def body(j, carry):
        base = w * PRC * PCH + j * PCH
        pltpu.make_async_copy(z_hbm.at[ia_v.at[j]], ra, semA).wait()
        pltpu.async_copy(zw_hbm.at[ib_v.at[j]], rb, semB)
        pltpu.sync_copy(ra, za_hbm.at[pl.ds(base, PCH)])
        pltpu.make_async_copy(zw_hbm.at[ib_v.at[j]], rb, semB).wait()
        pltpu.async_copy(z_hbm.at[ia_v.at[j + 1]], ra, semA)
        pltpu.sync_copy(rb, zb_hbm.at[pl.ds(base, PCH)])
        return carry

    lax.fori_loop(0, PRC, body, 0)
    pltpu.make_async_copy(z_hbm.at[ia_v.at[PRC]], ra, semA).wait()


_pair_kernel = functools.partial(
    pl.kernel, _pair_body,
    out_type=(jax.ShapeDtypeStruct((PP, 64), _f32),
              jax.ShapeDtypeStruct((PP, 64), _f32)),
    mesh=_mesh,
    compiler_params=_params,
    scratch_types=[
        pltpu.VMEM((PNCH, PCH), jnp.int32),
        pltpu.VMEM((PNCH, PCH), jnp.int32),
        pltpu.VMEM((PCH, 64), _f32),
        pltpu.VMEM((PCH, 64), _f32),
        pltpu.SemaphoreType.DMA,
        pltpu.SemaphoreType.DMA,
    ],
)()


def _dis(degp_ref):
    deg = degp_ref[0, :] + degp_ref[1, :] + 1.0
    return lax.rsqrt(deg)


def _prescale_body(x_ref, degp_ref, xp_ref):
    xp_ref[...] = x_ref[...] * _dis(degp_ref)[:, None]


def _prescale(x, degp, blk=1024):
    return pl.pallas_call(
        _prescale_body,
        grid=(pl.cdiv(N, blk),),
        in_specs=[
            pl.BlockSpec((blk, 128), lambda i: (i, 0)),
            pl.BlockSpec((NC, blk), lambda i: (0, i)),
        ],
        out_specs=pl.BlockSpec((blk, 128), lambda i: (i, 0)),
        out_shape=jax.ShapeDtypeStruct((N, 128), _f32),
    )(x, degp)


def _layer1_body(sp_ref, xp_ref, degp_ref, b1_ref, w1_ref, w2_ref, g_ref):
    dis = _dis(degp_ref)[:, None]
    m = (sp_ref[0] + sp_ref[1] + xp_ref[...]) * dis
    h = jax.nn.relu(jnp.dot(m, w1_ref[...],
                            preferred_element_type=_f32) + b1_ref[...])
    g_ref[...] = jnp.dot(h * dis, w2_ref[...], preferred_element_type=_f32)


def _layer1(sp, xp, degp, b1, W1, W2, blk=1024):
    return pl.pallas_call(
        _layer1_body,
        grid=(pl.cdiv(N, blk),),
        in_specs=[
            pl.BlockSpec((NC, blk, 128), lambda i: (0, i, 0)),
            pl.BlockSpec((blk, 128), lambda i: (i, 0)),
            pl.BlockSpec((NC, blk), lambda i: (0, i)),
            pl.BlockSpec((1, 128), lambda i: (0, 0)),
            pl.BlockSpec((128, 128), lambda i: (0, 0)),
            pl.BlockSpec((128, 64), lambda i: (0, 0)),
        ],
        out_specs=pl.BlockSpec((blk, 64), lambda i: (i, 0)),
        out_shape=jax.ShapeDtypeStruct((N, 64), _f32),
    )(sp, xp, degp, b1, W1, W2)


def _layer2_body(tp_ref, g_ref, degp_ref, b2_ref, w3_ref, z_ref, zw_ref):
    dis = _dis(degp_ref)[:, None]
    z = (tp_ref[0] + tp_ref[1] + g_ref[...]) * dis + b2_ref[...]
    z_ref[...] = z
    zw_ref[...] = z * w3_ref[...]


def _layer2(tp, g, degp, b2, w3row, blk=1024):
    return pl.pallas_call(
        _layer2_body,
        grid=(pl.cdiv(N, blk),),
        in_specs=[
            pl.BlockSpec((NC, blk, 64), lambda i: (0, i, 0)),
            pl.BlockSpec((blk, 64), lambda i: (i, 0)),
            pl.BlockSpec((NC, blk), lambda i: (0, i)),
            pl.BlockSpec((1, 64), lambda i: (0, 0)),
            pl.BlockSpec((1, 64), lambda i: (0, 0)),
        ],
        out_specs=[
            pl.BlockSpec((blk, 64), lambda i: (i, 0)),
            pl.BlockSpec((blk, 64), lambda i: (i, 0)),
        ],
        out_shape=[
            jax.ShapeDtypeStruct((N, 64), _f32),
            jax.ShapeDtypeStruct((N, 64), _f32),
        ],
    )(tp, g, degp, b2, w3row)


def _loss_body(za_ref, zb_ref, sc_ref, out_ref, acc_ref):
    i = pl.program_id(0)
    preds = jnp.sum(za_ref[...] * zb_ref[...], axis=1, keepdims=True) + sc_ref[1]
    sp = jnp.maximum(preds, 0.0) + jnp.log1p(jnp.exp(-jnp.abs(preds)))
    base = i * za_ref.shape[0]
    rows = base + lax.broadcasted_iota(jnp.int32, preds.shape, 0)
    labeled = (rows < 2).astype(_f32)
    contrib = jnp.sum(sp - labeled * preds)

    @pl.when(i == 0)
    def _init():
        acc_ref[0] = 0.0

    acc_ref[0] = acc_ref[0] + contrib

    @pl.when(i == pl.num_programs(0) - 1)
    def _fin():
        out_ref[0] = acc_ref[0]


def _loss_reduce(za, zb, b3, blk=1000):
    sc = jnp.stack([jnp.float32(0.0), b3[0]])
    out = pl.pallas_call(
        _loss_body,
        grid=(P2 // blk,),
        in_specs=[
            pl.BlockSpec((blk, 64), lambda i: (i, 0)),
            pl.BlockSpec((blk, 64), lambda i: (i, 0)),
            pl.BlockSpec(memory_space=pltpu.SMEM),
        ],
        out_specs=pl.BlockSpec(memory_space=pltpu.SMEM),
        out_shape=jax.ShapeDtypeStruct((1,), _f32),
        scratch_shapes=[pltpu.SMEM((1,), _f32)],
    )(za, zb, sc)
    return out[0] / P2


def kernel(x, ei, pos, neg, W1, b1, W2, b2, W3, b3):
    src3 = ei[0].reshape(NW, ENCH, ECH)
    dst3 = ei[1].reshape(NW, ENCH, ECH)
    padidx = jnp.arange(PP - P2, dtype=jnp.int32) % N
    extra = (jnp.arange(NW * PCH, dtype=jnp.int32) % N).reshape(NW, 1, PCH)
    ai = jnp.concatenate([pos[0], neg[0], padidx]).reshape(NW, PRC, PCH)
    bi = jnp.concatenate([pos[1], neg[1], padidx]).reshape(NW, PRC, PCH)
    ai3 = jnp.concatenate([ai, extra], axis=1)
    bi3 = jnp.concatenate([bi, extra], axis=1)
    z1 = jnp.zeros((NP,), _f32)
    z128 = jnp.zeros((NP, 128), _f32)
    z64 = jnp.zeros((NP, 64), _f32)

    degp = _deg_kernel(dst3, z1)
    xp = _prescale(x, degp)
    sp = _agg128(xp, src3, dst3, z128)
    g = _layer1(sp, xp, degp, b1.reshape(1, 128), W1, W2)
    tp = _agg64(g, src3, dst3, z64)
    z, zw = _layer2(tp, g, degp, b2.reshape(1, 64), W3.reshape(1, 64))
    za, zb = _pair_kernel(z, zw, ai3, bi3)
    return _loss_reduce(za, zb, b3)


# speedup vs baseline: 13.4175x; 1.0532x over previous
import functools

import jax
import jax.numpy as jnp
from jax import lax
from jax.experimental import pallas as pl
from jax.experimental.pallas import tpu as pltpu
from jax.experimental.pallas import tpu_sc as plsc

N = 10000
NP = 10240
E = 320000
P2 = 200000
NC = 2
NS = 16
NW = NC * NS
EPW = E // NW
ECH = 80
ENCH = EPW // ECH
PCH = 128
PRC = 50
PNCH = PRC + 1
PP = NW * PCH * PRC

_mesh = plsc.VectorSubcoreMesh(core_axis_name="c", subcore_axis_name="s")
_params = pltpu.CompilerParams(use_tc_tiling_on_sc=False)
_f32 = jnp.float32


def _wid():
    return lax.axis_index("s") * NC + lax.axis_index("c")


def _deg_body(dst_hbm, z1_hbm, out_hbm, di_v, ones_v, acc_sh, sem):
    c = lax.axis_index("c")
    s = lax.axis_index("s")
    stripe = NP // NS
    pltpu.sync_copy(z1_hbm.at[pl.ds(s * stripe, stripe)],
                    acc_sh.at[pl.ds(s * stripe, stripe)])
    pltpu.sync_copy(dst_hbm.at[_wid()], di_v)
    for i in range(ECH // 16):
        ones_v[pl.ds(i * 16, 16)] = jnp.ones((16,), _f32)
    plsc.subcore_barrier()

    def body(i, carry):
        for k in range(25):
            pltpu.async_copy(ones_v, acc_sh.at[di_v.at[i * 25 + k]], sem,
                             add=True)
        for k in range(25):
            pltpu.make_async_copy(ones_v, acc_sh.at[di_v.at[i * 25 + k]],
                                  sem).wait()
        return carry

    lax.fori_loop(0, ENCH // 25, body, 0)
    plsc.subcore_barrier()
    pltpu.sync_copy(acc_sh.at[pl.ds(s * stripe, stripe)],
                    out_hbm.at[c, pl.ds(s * stripe, stripe)])


_deg_kernel = functools.partial(
    pl.kernel, _deg_body,
    out_type=jax.ShapeDtypeStruct((NC, NP), _f32),
    mesh=_mesh,
    compiler_params=_params,
    scratch_types=[
        pltpu.VMEM((ENCH, ECH), jnp.int32),
        pltpu.VMEM((ECH,), _f32),
        pltpu.VMEM_SHARED((NP,), _f32),
        pltpu.SemaphoreType.DMA,
    ],
)()


def _agg_body(u_hbm, src_hbm, dst_hbm, zw_hbm, out_hbm,
              si_v, di_v, r0, r1, acc_sh, sem0, sem1):
    c = lax.axis_index("c")
    s = lax.axis_index("s")
    w = _wid()
    stripe = NP // NS
    pltpu.sync_copy(zw_hbm.at[pl.ds(s * stripe, stripe)],
                    acc_sh.at[pl.ds(s * stripe, stripe)])
    pltpu.sync_copy(src_hbm.at[w], si_v)
    pltpu.sync_copy(dst_hbm.at[w], di_v)
    plsc.subcore_barrier()

    pltpu.async_copy(u_hbm.at[si_v.at[0]], r0, sem0)

    def body(i, carry):
        j0 = 2 * i
        pltpu.make_async_copy(u_hbm.at[si_v.at[j0]], r0, sem0).wait()
        pltpu.async_copy(u_hbm.at[si_v.at[j0 + 1]], r1, sem1)
        pltpu.sync_copy(r0, acc_sh.at[di_v.at[j0]], add=True)
        pltpu.make_async_copy(u_hbm.at[si_v.at[j0 + 1]], r1, sem1).wait()
        pltpu.async_copy(u_hbm.at[si_v.at[j0 + 2]], r0, sem0)
        pltpu.sync_copy(r1, acc_sh.at[di_v.at[j0 + 1]], add=True)
        return carry

    lax.fori_loop(0, (ENCH - 1) // 2, body, 0)
    pltpu.make_async_copy(u_hbm.at[si_v.at[ENCH - 1]], r0, sem0).wait()
    pltpu.sync_copy(r0, acc_sh.at[di_v.at[ENCH - 1]], add=True)
    plsc.subcore_barrier()
    pltpu.sync_copy(acc_sh.at[pl.ds(s * stripe, stripe)],
                    out_hbm.at[c, pl.ds(s * stripe, stripe)])


def _make_agg(width):
    return functools.partial(
        pl.kernel, _agg_body,
        out_type=jax.ShapeDtypeStruct((NC, NP, width), _f32),
        mesh=_mesh,
        compiler_params=_params,
        scratch_types=[
            pltpu.VMEM((ENCH, ECH), jnp.int32),
            pltpu.VMEM((ENCH, ECH), jnp.int32),
            pltpu.VMEM((ECH, width), _f32),
            pltpu.VMEM((ECH, width), _f32),
            pltpu.VMEM_SHARED((NP, width), _f32),
            pltpu.SemaphoreType.DMA,
            pltpu.SemaphoreType.DMA,
        ],
    )()


_agg128 = _make_agg(128)
_agg64 = _make_agg(64)


def _dots(ra, rb, w3_v, pb):
    lanes = lax.iota(jnp.int32, 16)

    def gbody(gi, carry):
        rowi = lanes + gi * 16

        def kbody(k8, acc):
            for kk in range(8):
                k = k8 * 8 + kk
                col = jnp.full((16,), k, jnp.int32)
                ga = plsc.load_gather(ra, [rowi, col])
                gb = plsc.load_gather(rb, [rowi, col])
                gw = plsc.load_gather(w3_v, [col])
                acc = acc + ga * gb * gw
            return acc

        acc = lax.fori_loop(0, 8, kbody, jnp.zeros((16,), _f32))
        pb[pl.ds(gi * 16, 16)] = acc
        return carry

    lax.fori_loop(0, PCH // 16, gbody, 0)


def _pair_body(z_hbm, w3_hbm, ai_hbm, bi_hbm, pr_hbm,
               ia_v, ib_v, w3_v, ra0, rb0, ra1, rb1, pb0, pb1,
               sa0, sb0, sa1, sb1):
    w = _wid()
    pltpu.sync_copy(ai_hbm.at[w], ia_v)
    pltpu.sync_copy(bi_hbm.at[w], ib_v)
    pltpu.sync_copy(w3_hbm, w3_v)

    pltpu.async_copy(z_hbm.at[ia_v.at[0]], ra0, sa0)
    pltpu.async_copy(z_hbm.at[ib_v.at[0]], rb0, sb0)

    def body(i, carry):
        j0 = 2 * i
        base = w * PRC * PCH + j0 * PCH
        pltpu.make_async_copy(z_hbm.at[ia_v.at[j0]], ra0, sa0).wait()
        pltpu.make_async_copy(z_hbm.at[ib_v.at[j0]], rb0, sb0).wait()
        pltpu.async_copy(z_hbm.at[ia_v.at[j0 + 1]], ra1, sa1)
        pltpu.async_copy(z_hbm.at[ib_v.at[j0 + 1]], rb1, sb1)
        _dots(ra0, rb0, w3_v, pb0)
        pltpu.sync_copy(pb0, pr_hbm.at[pl.ds(base, PCH)])
        pltpu.make_async_copy(z_hbm.at[ia_v.at[j0 + 1]], ra1, sa1).wait()
        pltpu.make_async_copy(z_hbm.at[ib_v.at[j0 + 1]], rb1, sb1).wait()
        pltpu.async_copy(z_hbm.at[ia_v.at[j0 + 2]], ra0, sa0)
        pltpu.async_copy(z_hbm.at[ib_v.at[j0 + 2]], rb0, sb0)
        _dots(ra1, rb1, w3_v, pb1)
        pltpu.sync_copy(pb1, pr_hbm.at[pl.ds(base + PCH, PCH)])
        return carry

    lax.fori_loop(0, PRC // 2, body, 0)
    pltpu.make_async_copy(z_hbm.at[ia_v.at[PRC]], ra0, sa0).wait()
    pltpu.make_async_copy(z_hbm.at[ib_v.at[PRC]], rb0, sb0).wait()


_pair_kernel = functools.partial(
    pl.kernel, _pair_body,
    out_type=jax.ShapeDtypeStruct((PP,), _f32),
    mesh=_mesh,
    compiler_params=pltpu.CompilerParams(use_tc_tiling_on_sc=False,
                                         needs_layout_passes=False),
    scratch_types=[
        pltpu.VMEM((PNCH, PCH), jnp.int32),
        pltpu.VMEM((PNCH, PCH), jnp.int32),
        pltpu.VMEM((64,), _f32),
        pltpu.VMEM((PCH, 64), _f32),
        pltpu.VMEM((PCH, 64), _f32),
        pltpu.VMEM((PCH, 64), _f32),
        pltpu.VMEM((PCH, 64), _f32),
        pltpu.VMEM((PCH,), _f32),
        pltpu.VMEM((PCH,), _f32),
        pltpu.SemaphoreType.DMA,
        pltpu.SemaphoreType.DMA,
        pltpu.SemaphoreType.DMA,
        pltpu.SemaphoreType.DMA,
    ],
)()


def _dis(degp_ref):
    deg = degp_ref[0, :] + degp_ref[1, :] + 1.0
    return lax.rsqrt(deg)


def _prescale_body(x_ref, degp_ref, xp_ref):
    xp_ref[...] = x_ref[...] * _dis(degp_ref)[:, None]


def _prescale(x, degp, blk=1024):
    return pl.pallas_call(
        _prescale_body,
        grid=(pl.cdiv(N, blk),),
        in_specs=[
            pl.BlockSpec((blk, 128), lambda i: (i, 0)),
            pl.BlockSpec((NC, blk), lambda i: (0, i)),
        ],
        out_specs=pl.BlockSpec((blk, 128), lambda i: (i, 0)),
        out_shape=jax.ShapeDtypeStruct((N, 128), _f32),
    )(x, degp)


def _layer1_body(sp_ref, xp_ref, degp_ref, b1_ref, w1_ref, w2_ref, g_ref):
    dis = _dis(degp_ref)[:, None]
    m = (sp_ref[0] + sp_ref[1] + xp_ref[...]) * dis
    h = jax.nn.relu(jnp.dot(m, w1_ref[...],
                            preferred_element_type=_f32) + b1_ref[...])
    g_ref[...] = jnp.dot(h * dis, w2_ref[...], preferred_element_type=_f32)


def _layer1(sp, xp, degp, b1, W1, W2, blk=1024):
    return pl.pallas_call(
        _layer1_body,
        grid=(pl.cdiv(N, blk),),
        in_specs=[
            pl.BlockSpec((NC, blk, 128), lambda i: (0, i, 0)),
            pl.BlockSpec((blk, 128), lambda i: (i, 0)),
            pl.BlockSpec((NC, blk), lambda i: (0, i)),
            pl.BlockSpec((1, 128), lambda i: (0, 0)),
            pl.BlockSpec((128, 128), lambda i: (0, 0)),
            pl.BlockSpec((128, 64), lambda i: (0, 0)),
        ],
        out_specs=pl.BlockSpec((blk, 64), lambda i: (i, 0)),
        out_shape=jax.ShapeDtypeStruct((N, 64), _f32),
    )(sp, xp, degp, b1, W1, W2)


def _layer2_body(tp_ref, g_ref, degp_ref, b2_ref, z_ref):
    dis = _dis(degp_ref)[:, None]
    z_ref[...] = (tp_ref[0] + tp_ref[1] + g_ref[...]) * dis + b2_ref[...]


def _layer2(tp, g, degp, b2, blk=1024):
    return pl.pallas_call(
        _layer2_body,
        grid=(pl.cdiv(N, blk),),
        in_specs=[
            pl.BlockSpec((NC, blk, 64), lambda i: (0, i, 0)),
            pl.BlockSpec((blk, 64), lambda i: (i, 0)),
            pl.BlockSpec((NC, blk), lambda i: (0, i)),
            pl.BlockSpec((1, 64), lambda i: (0, 0)),
        ],
        out_specs=pl.BlockSpec((blk, 64), lambda i: (i, 0)),
        out_shape=jax.ShapeDtypeStruct((N, 64), _f32),
    )(tp, g, degp, b2)


def _loss_body(pr_ref, sc_ref, out_ref):
    preds = pr_ref[...] + sc_ref[0]
    sp = jnp.maximum(preds, 0.0) + jnp.log1p(jnp.exp(-jnp.abs(preds)))
    flat = (lax.broadcasted_iota(jnp.int32, preds.shape, 0) * 128
            + lax.broadcasted_iota(jnp.int32, preds.shape, 1))
    labeled = (flat < 2).astype(_f32)
    valid = (flat < P2).astype(_f32)
    out_ref[0] = jnp.sum((sp - labeled * preds) * valid)


def _loss_reduce(preds2d, b3):
    out = pl.pallas_call(
        _loss_body,
        in_specs=[
            pl.BlockSpec((PP // 128, 128), lambda: (0, 0)),
            pl.BlockSpec(memory_space=pltpu.SMEM),
        ],
        out_specs=pl.BlockSpec(memory_space=pltpu.SMEM),
        out_shape=jax.ShapeDtypeStruct((1,), _f32),
    )(preds2d, b3)
    return out[0] / P2


def kernel(x, ei, pos, neg, W1, b1, W2, b2, W3, b3):
    src3 = ei[0].reshape(NW, ENCH, ECH)
    dst3 = ei[1].reshape(NW, ENCH, ECH)
    padidx = jnp.arange(PP - P2, dtype=jnp.int32) % N
    extra = (jnp.arange(NW * PCH, dtype=jnp.int32) % N).reshape(NW, 1, PCH)
    ai = jnp.concatenate([pos[0], neg[0], padidx]).reshape(NW, PRC, PCH)
    bi = jnp.concatenate([pos[1], neg[1], padidx]).reshape(NW, PRC, PCH)
    ai3 = jnp.concatenate([ai, extra], axis=1)
    bi3 = jnp.concatenate([bi, extra], axis=1)
    z1 = jnp.zeros((NP,), _f32)
    z128 = jnp.zeros((NP, 128), _f32)
    z64 = jnp.zeros((NP, 64), _f32)

    degp = _deg_kernel(dst3, z1)
    xp = _prescale(x, degp)
    sp = _agg128(xp, src3, dst3, z128)
    g = _layer1(sp, xp, degp, b1.reshape(1, 128), W1, W2)
    tp = _agg64(g, src3, dst3, z64)
    z = _layer2(tp, g, degp, b2.reshape(1, 64))
    preds = _pair_kernel(z, W3[:, 0], ai3, bi3)
    return _loss_reduce(preds.reshape(PP // 128, 128), b3)


# speedup vs baseline: 24.1963x; 1.8033x over previous
import functools

import jax
import jax.numpy as jnp
from jax import lax
from jax.experimental import pallas as pl
from jax.experimental.pallas import tpu as pltpu
from jax.experimental.pallas import tpu_sc as plsc

N = 10000
NP = 10240
E = 320000
P2 = 200000
NC = 2
NS = 16
NW = NC * NS
EPW = E // NW
ECH = 80
ENCH = EPW // ECH
PCH = 128
PRC = 50
PNCH = PRC + 1
PP = NW * PCH * PRC

_mesh = plsc.VectorSubcoreMesh(core_axis_name="c", subcore_axis_name="s")
_params = pltpu.CompilerParams(use_tc_tiling_on_sc=False)
_f32 = jnp.float32


def _wid():
    return lax.axis_index("s") * NC + lax.axis_index("c")


def _deg_body(dst_hbm, z1_hbm, out_hbm, di_v, ones_v, acc_sh, sem):
    c = lax.axis_index("c")
    s = lax.axis_index("s")
    stripe = NP // NS
    pltpu.sync_copy(z1_hbm.at[pl.ds(s * stripe, stripe)],
                    acc_sh.at[pl.ds(s * stripe, stripe)])
    pltpu.sync_copy(dst_hbm.at[_wid()], di_v)
    for i in range(ECH // 16):
        ones_v[pl.ds(i * 16, 16)] = jnp.ones((16,), _f32)
    plsc.subcore_barrier()

    def body(i, carry):
        for k in range(25):
            pltpu.async_copy(ones_v, acc_sh.at[di_v.at[i * 25 + k]], sem,
                             add=True)
        for k in range(25):
            pltpu.make_async_copy(ones_v, acc_sh.at[di_v.at[i * 25 + k]],
                                  sem).wait()
        return carry

    lax.fori_loop(0, ENCH // 25, body, 0)
    plsc.subcore_barrier()
    pltpu.sync_copy(acc_sh.at[pl.ds(s * stripe, stripe)],
                    out_hbm.at[c, pl.ds(s * stripe, stripe)])


_deg_kernel = functools.partial(
    pl.kernel, _deg_body,
    out_type=jax.ShapeDtypeStruct((NC, NP), _f32),
    mesh=_mesh,
    compiler_params=_params,
    scratch_types=[
        pltpu.VMEM((ENCH, ECH), jnp.int32),
        pltpu.VMEM((ECH,), _f32),
        pltpu.VMEM_SHARED((NP,), _f32),
        pltpu.SemaphoreType.DMA,
    ],
)()


def _agg_body(u_hbm, src_hbm, dst_hbm, zw_hbm, out_hbm,
              si_v, di_v, r0, r1, acc_sh, sem0, sem1):
    c = lax.axis_index("c")
    s = lax.axis_index("s")
    w = _wid()
    stripe = NP // NS
    pltpu.sync_copy(zw_hbm.at[pl.ds(s * stripe, stripe)],
                    acc_sh.at[pl.ds(s * stripe, stripe)])
    pltpu.sync_copy(src_hbm.at[w], si_v)
    pltpu.sync_copy(dst_hbm.at[w], di_v)
    plsc.subcore_barrier()

    pltpu.async_copy(u_hbm.at[si_v.at[0]], r0, sem0)

    def body(i, carry):
        j0 = 2 * i
        pltpu.make_async_copy(u_hbm.at[si_v.at[j0]], r0, sem0).wait()
        pltpu.async_copy(u_hbm.at[si_v.at[j0 + 1]], r1, sem1)
        pltpu.sync_copy(r0, acc_sh.at[di_v.at[j0]], add=True)
        pltpu.make_async_copy(u_hbm.at[si_v.at[j0 + 1]], r1, sem1).wait()
        pltpu.async_copy(u_hbm.at[si_v.at[j0 + 2]], r0, sem0)
        pltpu.sync_copy(r1, acc_sh.at[di_v.at[j0 + 1]], add=True)
        return carry

    lax.fori_loop(0, (ENCH - 1) // 2, body, 0)
    pltpu.make_async_copy(u_hbm.at[si_v.at[ENCH - 1]], r0, sem0).wait()
    pltpu.sync_copy(r0, acc_sh.at[di_v.at[ENCH - 1]], add=True)
    plsc.subcore_barrier()
    pltpu.sync_copy(acc_sh.at[pl.ds(s * stripe, stripe)],
                    out_hbm.at[c, pl.ds(s * stripe, stripe)])


def _make_agg(width):
    return functools.partial(
        pl.kernel, _agg_body,
        out_type=jax.ShapeDtypeStruct((NC, NP, width), _f32),
        mesh=_mesh,
        compiler_params=_params,
        scratch_types=[
            pltpu.VMEM((ENCH, ECH), jnp.int32),
            pltpu.VMEM((ENCH, ECH), jnp.int32),
            pltpu.VMEM((ECH, width), _f32),
            pltpu.VMEM((ECH, width), _f32),
            pltpu.VMEM_SHARED((NP, width), _f32),
            pltpu.SemaphoreType.DMA,
            pltpu.SemaphoreType.DMA,
        ],
    )()


_agg128 = _make_agg(128)
_agg64 = _make_agg(64)


def _perm16(x, idx):
    dnums = lax.GatherDimensionNumbers(
        offset_dims=(), collapsed_slice_dims=(0,), start_index_map=(0,))
    return lax.gather(x, idx[:, None], dnums, (1,),
                      mode=lax.GatherScatterMode.PROMISE_IN_BOUNDS)


def _dots(ra, rb, pb):
    lanes = lax.iota(jnp.int32, 16)
    perms = [lanes ^ sh for sh in (8, 4, 2, 1)]

    def gbody(g, carry):
        res = jnp.zeros((16,), _f32)
        for u in range(16):
            p = g * 16 + u
            acc = ra[p, pl.ds(0, 16)] * rb[p, pl.ds(0, 16)]
            for cc in range(1, 4):
                acc = acc + ra[p, pl.ds(cc * 16, 16)] * rb[p, pl.ds(cc * 16, 16)]
            for pm in perms:
                acc = acc + _perm16(acc, pm)
            res = jnp.where(lanes == u, acc, res)
        pb[pl.ds(g * 16, 16)] = res
        return carry

    lax.fori_loop(0, PCH // 16, gbody, 0)


def _pair_body(z_hbm, zw_hbm, ai_hbm, bi_hbm, pr_hbm,
               ia_v, ib_v, ra0, rb0, ra1, rb1, pb0, pb1,
               sa0, sb0, sa1, sb1):
    w = _wid()
    pltpu.sync_copy(ai_hbm.at[w], ia_v)
    pltpu.sync_copy(bi_hbm.at[w], ib_v)

    pltpu.async_copy(z_hbm.at[ia_v.at[0]], ra0, sa0)
    pltpu.async_copy(zw_hbm.at[ib_v.at[0]], rb0, sb0)

    def body(i, carry):
        j0 = 2 * i
        base = w * PRC * PCH + j0 * PCH
        pltpu.make_async_copy(z_hbm.at[ia_v.at[j0]], ra0, sa0).wait()
        pltpu.make_async_copy(zw_hbm.at[ib_v.at[j0]], rb0, sb0).wait()
        pltpu.async_copy(z_hbm.at[ia_v.at[j0 + 1]], ra1, sa1)
        pltpu.async_copy(zw_hbm.at[ib_v.at[j0 + 1]], rb1, sb1)
        _dots(ra0, rb0, pb0)
        pltpu.sync_copy(pb0, pr_hbm.at[pl.ds(base, PCH)])
        pltpu.make_async_copy(z_hbm.at[ia_v.at[j0 + 1]], ra1, sa1).wait()
        pltpu.make_async_copy(zw_hbm.at[ib_v.at[j0 + 1]], rb1, sb1).wait()
        pltpu.async_copy(z_hbm.at[ia_v.at[j0 + 2]], ra0, sa0)
        pltpu.async_copy(zw_hbm.at[ib_v.at[j0 + 2]], rb0, sb0)
        _dots(ra1, rb1, pb1)
        pltpu.sync_copy(pb1, pr_hbm.at[pl.ds(base + PCH, PCH)])
        return carry

    lax.fori_loop(0, PRC // 2, body, 0)
    pltpu.make_async_copy(z_hbm.at[ia_v.at[PRC]], ra0, sa0).wait()
    pltpu.make_async_copy(zw_hbm.at[ib_v.at[PRC]], rb0, sb0).wait()


_pair_kernel = functools.partial(
    pl.kernel, _pair_body,
    out_type=jax.ShapeDtypeStruct((PP,), _f32),
    mesh=_mesh,
    compiler_params=_params,
    scratch_types=[
        pltpu.VMEM((PNCH, PCH), jnp.int32),
        pltpu.VMEM((PNCH, PCH), jnp.int32),
        pltpu.VMEM((PCH, 64), _f32),
        pltpu.VMEM((PCH, 64), _f32),
        pltpu.VMEM((PCH, 64), _f32),
        pltpu.VMEM((PCH, 64), _f32),
        pltpu.VMEM((PCH,), _f32),
        pltpu.VMEM((PCH,), _f32),
        pltpu.SemaphoreType.DMA,
        pltpu.SemaphoreType.DMA,
        pltpu.SemaphoreType.DMA,
        pltpu.SemaphoreType.DMA,
    ],
)()


def _dis(degp_ref):
    deg = degp_ref[0, :] + degp_ref[1, :] + 1.0
    return lax.rsqrt(deg)


def _prescale_body(x_ref, degp_ref, xp_ref):
    xp_ref[...] = x_ref[...] * _dis(degp_ref)[:, None]


def _prescale(x, degp, blk=1024):
    return pl.pallas_call(
        _prescale_body,
        grid=(pl.cdiv(N, blk),),
        in_specs=[
            pl.BlockSpec((blk, 128), lambda i: (i, 0)),
            pl.BlockSpec((NC, blk), lambda i: (0, i)),
        ],
        out_specs=pl.BlockSpec((blk, 128), lambda i: (i, 0)),
        out_shape=jax.ShapeDtypeStruct((N, 128), _f32),
    )(x, degp)


def _layer1_body(sp_ref, xp_ref, degp_ref, b1_ref, w1_ref, w2_ref, g_ref):
    dis = _dis(degp_ref)[:, None]
    m = (sp_ref[0] + sp_ref[1] + xp_ref[...]) * dis
    h = jax.nn.relu(jnp.dot(m, w1_ref[...],
                            preferred_element_type=_f32) + b1_ref[...])
    g_ref[...] = jnp.dot(h * dis, w2_ref[...], preferred_element_type=_f32)


def _layer1(sp, xp, degp, b1, W1, W2, blk=1024):
    return pl.pallas_call(
        _layer1_body,
        grid=(pl.cdiv(N, blk),),
        in_specs=[
            pl.BlockSpec((NC, blk, 128), lambda i: (0, i, 0)),
            pl.BlockSpec((blk, 128), lambda i: (i, 0)),
            pl.BlockSpec((NC, blk), lambda i: (0, i)),
            pl.BlockSpec((1, 128), lambda i: (0, 0)),
            pl.BlockSpec((128, 128), lambda i: (0, 0)),
            pl.BlockSpec((128, 64), lambda i: (0, 0)),
        ],
        out_specs=pl.BlockSpec((blk, 64), lambda i: (i, 0)),
        out_shape=jax.ShapeDtypeStruct((N, 64), _f32),
    )(sp, xp, degp, b1, W1, W2)


def _layer2_body(tp_ref, g_ref, degp_ref, b2_ref, w3_ref, z_ref, zw_ref):
    dis = _dis(degp_ref)[:, None]
    z = (tp_ref[0] + tp_ref[1] + g_ref[...]) * dis + b2_ref[...]
    z_ref[...] = z
    zw_ref[...] = z * w3_ref[...]


def _layer2(tp, g, degp, b2, w3row, blk=1024):
    return pl.pallas_call(
        _layer2_body,
        grid=(pl.cdiv(N, blk),),
        in_specs=[
            pl.BlockSpec((NC, blk, 64), lambda i: (0, i, 0)),
            pl.BlockSpec((blk, 64), lambda i: (i, 0)),
            pl.BlockSpec((NC, blk), lambda i: (0, i)),
            pl.BlockSpec((1, 64), lambda i: (0, 0)),
            pl.BlockSpec((1, 64), lambda i: (0, 0)),
        ],
        out_specs=[
            pl.BlockSpec((blk, 64), lambda i: (i, 0)),
            pl.BlockSpec((blk, 64), lambda i: (i, 0)),
        ],
        out_shape=[
            jax.ShapeDtypeStruct((N, 64), _f32),
            jax.ShapeDtypeStruct((N, 64), _f32),
        ],
    )(tp, g, degp, b2, w3row)


def _loss_body(pr_ref, sc_ref, out_ref):
    preds = pr_ref[...] + sc_ref[0]
    sp = jnp.maximum(preds, 0.0) + jnp.log1p(jnp.exp(-jnp.abs(preds)))
    flat = (lax.broadcasted_iota(jnp.int32, preds.shape, 0) * 128
            + lax.broadcasted_iota(jnp.int32, preds.shape, 1))
    labeled = (flat < 2).astype(_f32)
    valid = (flat < P2).astype(_f32)
    out_ref[0] = jnp.sum((sp - labeled * preds) * valid)


def _loss_reduce(preds2d, b3):
    out = pl.pallas_call(
        _loss_body,
        in_specs=[
            pl.BlockSpec((PP // 128, 128), lambda: (0, 0)),
            pl.BlockSpec(memory_space=pltpu.SMEM),
        ],
        out_specs=pl.BlockSpec(memory_space=pltpu.SMEM),
        out_shape=jax.ShapeDtypeStruct((1,), _f32),
    )(preds2d, b3)
    return out[0] / P2


def kernel(x, ei, pos, neg, W1, b1, W2, b2, W3, b3):
    src3 = ei[0].reshape(NW, ENCH, ECH)
    dst3 = ei[1].reshape(NW, ENCH, ECH)
    padidx = jnp.arange(PP - P2, dtype=jnp.int32) % N
    extra = (jnp.arange(NW * PCH, dtype=jnp.int32) % N).reshape(NW, 1, PCH)
    ai = jnp.concatenate([pos[0], neg[0], padidx]).reshape(NW, PRC, PCH)
    bi = jnp.concatenate([pos[1], neg[1], padidx]).reshape(NW, PRC, PCH)
    ai3 = jnp.concatenate([ai, extra], axis=1)
    bi3 = jnp.concatenate([bi, extra], axis=1)
    z1 = jnp.zeros((NP,), _f32)
    z128 = jnp.zeros((NP, 128), _f32)
    z64 = jnp.zeros((NP, 64), _f32)

    degp = _deg_kernel(dst3, z1)
    xp = _prescale(x, degp)
    sp = _agg128(xp, src3, dst3, z128)
    g = _layer1(sp, xp, degp, b1.reshape(1, 128), W1, W2)
    tp = _agg64(g, src3, dst3, z64)
    z, zw = _layer2(tp, g, degp, b2.reshape(1, 64), W3.reshape(1, 64))
    preds = _pair_kernel(z, zw, ai3, bi3)
    return _loss_reduce(preds.reshape(PP // 128, 128), b3)


# speedup vs baseline: 24.1975x; 1.0001x over previous
import functools

import jax
import jax.numpy as jnp
from jax import lax
from jax.experimental import pallas as pl
from jax.experimental.pallas import tpu as pltpu
from jax.experimental.pallas import tpu_sc as plsc

N = 10000
NP = 10240
E = 320000
P2 = 200000
NC = 2
NS = 16
NW = NC * NS
EPW = E // NW
ECH = 80
ENCH = EPW // ECH
NPA = NP
PCH = 128
PRC = 50
PNCH = PRC + 1
PP = NW * PCH * PRC

_mesh = plsc.VectorSubcoreMesh(core_axis_name="c", subcore_axis_name="s")
_params = pltpu.CompilerParams(use_tc_tiling_on_sc=False)
_f32 = jnp.float32


def _wid():
    return lax.axis_index("s") * NC + lax.axis_index("c")


def _deg_body(dst_hbm, z1_hbm, out_hbm, di_v, ones_v, acc_sh, sem):
    c = lax.axis_index("c")
    s = lax.axis_index("s")
    stripe = NP // NS
    pltpu.sync_copy(z1_hbm.at[pl.ds(s * stripe, stripe)],
                    acc_sh.at[pl.ds(s * stripe, stripe)])
    pltpu.sync_copy(dst_hbm.at[_wid()], di_v)
    for i in range(ECH // 16):
        ones_v[pl.ds(i * 16, 16)] = jnp.ones((16,), _f32)
    plsc.subcore_barrier()

    def body(i, carry):
        for k in range(25):
            pltpu.async_copy(ones_v, acc_sh.at[di_v.at[i * 25 + k]], sem,
                             add=True)
        for k in range(25):
            pltpu.make_async_copy(ones_v, acc_sh.at[di_v.at[i * 25 + k]],
                                  sem).wait()
        return carry

    lax.fori_loop(0, ENCH // 25, body, 0)
    plsc.subcore_barrier()
    pltpu.sync_copy(acc_sh.at[pl.ds(s * stripe, stripe)],
                    out_hbm.at[c, pl.ds(s * stripe, stripe)])


_deg_kernel = functools.partial(
    pl.kernel, _deg_body,
    out_type=jax.ShapeDtypeStruct((NC, NP), _f32),
    mesh=_mesh,
    compiler_params=_params,
    scratch_types=[
        pltpu.VMEM((ENCH, ECH), jnp.int32),
        pltpu.VMEM((ECH,), _f32),
        pltpu.VMEM_SHARED((NP,), _f32),
        pltpu.SemaphoreType.DMA,
    ],
)()


def _agg_body(u_hbm, src_hbm, dst_hbm, zw_hbm, out_hbm,
              si_v, di_v, r0, r1, acc_sh, sem0, sem1):
    c = lax.axis_index("c")
    s = lax.axis_index("s")
    w = _wid()
    stripe = NPA // NS
    pltpu.sync_copy(zw_hbm.at[pl.ds(s * stripe, stripe)],
                    acc_sh.at[pl.ds(s * stripe, stripe)])
    pltpu.sync_copy(src_hbm.at[w], si_v)
    pltpu.sync_copy(dst_hbm.at[w], di_v)
    plsc.subcore_barrier()

    pltpu.async_copy(u_hbm.at[si_v.at[0]], r0, sem0)

    def body(i, carry):
        j0 = 2 * i
        pltpu.make_async_copy(u_hbm.at[si_v.at[j0]], r0, sem0).wait()
        pltpu.async_copy(u_hbm.at[si_v.at[j0 + 1]], r1, sem1)
        pltpu.sync_copy(r0, acc_sh.at[di_v.at[j0]], add=True)
        pltpu.make_async_copy(u_hbm.at[si_v.at[j0 + 1]], r1, sem1).wait()
        pltpu.async_copy(u_hbm.at[si_v.at[j0 + 2]], r0, sem0)
        pltpu.sync_copy(r1, acc_sh.at[di_v.at[j0 + 1]], add=True)
        return carry

    lax.fori_loop(0, (ENCH - 1) // 2, body, 0)
    pltpu.make_async_copy(u_hbm.at[si_v.at[ENCH - 1]], r0, sem0).wait()
    pltpu.sync_copy(r0, acc_sh.at[di_v.at[ENCH - 1]], add=True)
    plsc.subcore_barrier()
    pltpu.sync_copy(acc_sh.at[pl.ds(s * stripe, stripe)],
                    out_hbm.at[c, pl.ds(s * stripe, stripe)])


def _make_agg(width):
    return functools.partial(
        pl.kernel, _agg_body,
        out_type=jax.ShapeDtypeStruct((NC, NPA, width), _f32),
        mesh=_mesh,
        compiler_params=_params,
        scratch_types=[
            pltpu.VMEM((ENCH, ECH), jnp.int32),
            pltpu.VMEM((ENCH, ECH), jnp.int32),
            pltpu.VMEM((ECH, width), _f32),
            pltpu.VMEM((ECH, width), _f32),
            pltpu.VMEM_SHARED((NPA, width), _f32),
            pltpu.SemaphoreType.DMA,
            pltpu.SemaphoreType.DMA,
        ],
    )()


_agg128 = _make_agg(128)
_agg64 = _make_agg(64)


def _perm16(x, idx):
    dnums = lax.GatherDimensionNumbers(
        offset_dims=(), collapsed_slice_dims=(0,), start_index_map=(0,))
    return lax.gather(x, idx[:, None], dnums, (1,),
                      mode=lax.GatherScatterMode.PROMISE_IN_BOUNDS)


def _dots(ra, rb, pb):
    lanes = lax.iota(jnp.int32, 16)
    perms = [lanes ^ sh for sh in (8, 4, 2, 1)]

    def gbody(g, carry):
        res = jnp.zeros((16,), _f32)
        for u in range(16):
            p = g * 16 + u
            acc = ra[p, pl.ds(0, 16)] * rb[p, pl.ds(0, 16)]
            for cc in range(1, 4):
                acc = acc + ra[p, pl.ds(cc * 16, 16)] * rb[p, pl.ds(cc * 16, 16)]
            for pm in perms:
                acc = acc + _perm16(acc, pm)
            res = jnp.where(lanes == u, acc, res)
        pb[pl.ds(g * 16, 16)] = res
        return carry

    lax.fori_loop(0, PCH // 16, gbody, 0)


def _pair_body(z_hbm, zw_hbm, ai_hbm, bi_hbm, pr_hbm,
               ia_v, ib_v, ra0, rb0, ra1, rb1, pb0, pb1,
               sa0, sb0, sa1, sb1):
    w = _wid()
    pltpu.sync_copy(ai_hbm.at[w], ia_v)
    pltpu.sync_copy(bi_hbm.at[w], ib_v)

    pltpu.async_copy(z_hbm.at[ia_v.at[0]], ra0, sa0)
    pltpu.async_copy(zw_hbm.at[ib_v.at[0]], rb0, sb0)

    def body(i, carry):
        j0 = 2 * i
        base = w * PRC * PCH + j0 * PCH
        pltpu.make_async_copy(z_hbm.at[ia_v.at[j0]], ra0, sa0).wait()
        pltpu.make_async_copy(zw_hbm.at[ib_v.at[j0]], rb0, sb0).wait()
        pltpu.async_copy(z_hbm.at[ia_v.at[j0 + 1]], ra1, sa1)
        pltpu.async_copy(zw_hbm.at[ib_v.at[j0 + 1]], rb1, sb1)
        _dots(ra0, rb0, pb0)
        pltpu.sync_copy(pb0, pr_hbm.at[pl.ds(base, PCH)])
        pltpu.make_async_copy(z_hbm.at[ia_v.at[j0 + 1]], ra1, sa1).wait()
        pltpu.make_async_copy(zw_hbm.at[ib_v.at[j0 + 1]], rb1, sb1).wait()
        pltpu.async_copy(z_hbm.at[ia_v.at[j0 + 2]], ra0, sa0)
        pltpu.async_copy(zw_hbm.at[ib_v.at[j0 + 2]], rb0, sb0)
        _dots(ra1, rb1, pb1)
        pltpu.sync_copy(pb1, pr_hbm.at[pl.ds(base + PCH, PCH)])
        return carry

    lax.fori_loop(0, PRC // 2, body, 0)
    pltpu.make_async_copy(z_hbm.at[ia_v.at[PRC]], ra0, sa0).wait()
    pltpu.make_async_copy(zw_hbm.at[ib_v.at[PRC]], rb0, sb0).wait()


_pair_kernel = functools.partial(
    pl.kernel, _pair_body,
    out_type=jax.ShapeDtypeStruct((PP,), _f32),
    mesh=_mesh,
    compiler_params=_params,
    scratch_types=[
        pltpu.VMEM((PNCH, PCH), jnp.int32),
        pltpu.VMEM((PNCH, PCH), jnp.int32),
        pltpu.VMEM((PCH, 64), _f32),
        pltpu.VMEM((PCH, 64), _f32),
        pltpu.VMEM((PCH, 64), _f32),
        pltpu.VMEM((PCH, 64), _f32),
        pltpu.VMEM((PCH,), _f32),
        pltpu.VMEM((PCH,), _f32),
        pltpu.SemaphoreType.DMA,
        pltpu.SemaphoreType.DMA,
        pltpu.SemaphoreType.DMA,
        pltpu.SemaphoreType.DMA,
    ],
)()


def _dis(degp_ref):
    deg = degp_ref[0, :] + degp_ref[1, :] + 1.0
    return lax.rsqrt(deg)


def _prescale_body(x_ref, degp_ref, xp_ref):
    xp_ref[...] = x_ref[...] * _dis(degp_ref)[:, None]


def _prescale(x, degp, blk=1024):
    return pl.pallas_call(
        _prescale_body,
        grid=(pl.cdiv(N, blk),),
        in_specs=[
            pl.BlockSpec((blk, 128), lambda i: (i, 0)),
            pl.BlockSpec((NC, blk), lambda i: (0, i)),
        ],
        out_specs=pl.BlockSpec((blk, 128), lambda i: (i, 0)),
        out_shape=jax.ShapeDtypeStruct((N, 128), _f32),
    )(x, degp)


def _layer1_body(sp_ref, xp_ref, degp_ref, b1_ref, w1_ref, w2_ref, g_ref):
    dis = _dis(degp_ref)[:, None]
    m = (sp_ref[0] + sp_ref[1] + xp_ref[...]) * dis
    h = jax.nn.relu(jnp.dot(m, w1_ref[...],
                            preferred_element_type=_f32) + b1_ref[...])
    g_ref[...] = jnp.dot(h * dis, w2_ref[...], preferred_element_type=_f32)


def _layer1(sp, xp, degp, b1, W1, W2, blk=1024):
    return pl.pallas_call(
        _layer1_body,
        grid=(pl.cdiv(N, blk),),
        in_specs=[
            pl.BlockSpec((NC, blk, 128), lambda i: (0, i, 0)),
            pl.BlockSpec((blk, 128), lambda i: (i, 0)),
            pl.BlockSpec((NC, blk), lambda i: (0, i)),
            pl.BlockSpec((1, 128), lambda i: (0, 0)),
            pl.BlockSpec((128, 128), lambda i: (0, 0)),
            pl.BlockSpec((128, 64), lambda i: (0, 0)),
        ],
        out_specs=pl.BlockSpec((blk, 64), lambda i: (i, 0)),
        out_shape=jax.ShapeDtypeStruct((N, 64), _f32),
    )(sp, xp, degp, b1, W1, W2)


def _layer2_body(tp_ref, g_ref, degp_ref, b2_ref, w3_ref, z_ref, zw_ref):
    dis = _dis(degp_ref)[:, None]
    z = (tp_ref[0] + tp_ref[1] + g_ref[...]) * dis + b2_ref[...]
    z_ref[...] = z
    zw_ref[...] = z * w3_ref[...]


def _layer2(tp, g, degp, b2, w3row, blk=1024):
    return pl.pallas_call(
        _layer2_body,
        grid=(pl.cdiv(N, blk),),
        in_specs=[
            pl.BlockSpec((NC, blk, 64), lambda i: (0, i, 0)),
            pl.BlockSpec((blk, 64), lambda i: (i, 0)),
            pl.BlockSpec((NC, blk), lambda i: (0, i)),
            pl.BlockSpec((1, 64), lambda i: (0, 0)),
            pl.BlockSpec((1, 64), lambda i: (0, 0)),
        ],
        out_specs=[
            pl.BlockSpec((blk, 64), lambda i: (i, 0)),
            pl.BlockSpec((blk, 64), lambda i: (i, 0)),
        ],
        out_shape=[
            jax.ShapeDtypeStruct((N, 64), _f32),
            jax.ShapeDtypeStruct((N, 64), _f32),
        ],
    )(tp, g, degp, b2, w3row)


def _loss_body(pr_ref, sc_ref, out_ref):
    preds = pr_ref[...] + sc_ref[0]
    sp = jnp.maximum(preds, 0.0) + jnp.log1p(jnp.exp(-jnp.abs(preds)))
    flat = (lax.broadcasted_iota(jnp.int32, preds.shape, 0) * 128
            + lax.broadcasted_iota(jnp.int32, preds.shape, 1))
    labeled = (flat < 2).astype(_f32)
    valid = (flat < P2).astype(_f32)
    out_ref[0] = jnp.sum((sp - labeled * preds) * valid)


def _loss_reduce(preds2d, b3):
    out = pl.pallas_call(
        _loss_body,
        in_specs=[
            pl.BlockSpec((PP // 128, 128), lambda: (0, 0)),
            pl.BlockSpec(memory_space=pltpu.SMEM),
        ],
        out_specs=pl.BlockSpec(memory_space=pltpu.SMEM),
        out_shape=jax.ShapeDtypeStruct((1,), _f32),
    )(preds2d, b3)
    return out[0] / P2


def kernel(x, ei, pos, neg, W1, b1, W2, b2, W3, b3):
    src3 = ei[0].reshape(NW, ENCH, ECH)
    dst3 = ei[1].reshape(NW, ENCH, ECH)
    padidx = jnp.arange(PP - P2, dtype=jnp.int32) % N
    extra = (jnp.arange(NW * PCH, dtype=jnp.int32) % N).reshape(NW, 1, PCH)
    ai = jnp.concatenate([pos[0], neg[0], padidx]).reshape(NW, PRC, PCH)
    bi = jnp.concatenate([pos[1], neg[1], padidx]).reshape(NW, PRC, PCH)
    ai3 = jnp.concatenate([ai, extra], axis=1)
    bi3 = jnp.concatenate([bi, extra], axis=1)
    z1 = jnp.zeros((NP,), _f32)
    z128 = jnp.zeros((NP, 128), _f32)
    z64 = jnp.zeros((NP, 64), _f32)

    degp = _deg_kernel(dst3, z1)
    xp = _prescale(x, degp)
    sp = _agg128(xp, src3, dst3, z128)
    g = _layer1(sp, xp, degp, b1.reshape(1, 128), W1, W2)
    tp = _agg64(g, src3, dst3, z64)
    z, zw = _layer2(tp, g, degp, b2.reshape(1, 64), W3.reshape(1, 64))
    preds = _pair_kernel(z, zw, ai3, bi3)
    return _loss_reduce(preds.reshape(PP // 128, 128), b3)
